# baseline (speedup 1.0000x reference)
import functools

import jax
import jax.numpy as jnp
from jax import lax
from jax.experimental import pallas as pl
from jax.experimental.pallas import tpu as pltpu
from jax.experimental.pallas import tpu_sc as plsc

_N = 10000
_E = 320000
_H = 128
_LANES = 16
_NC = 2
_NS = 16
_NW = _NC * _NS
_EPT = _E // _NW
_K = 80
_NCHUNK = _EPT // _K
_NPAD = 10240
_STRIPE = _NPAD // _NS
_ZROWS = 128
def _edge_body(a_hbm, b_hbm, p_hbm,
               s_out,
               s_sh,
               idx0, idx1,
               ar0, ar1, br0, br1,
               sa0, sa1, sb0, sb1, ss0, ss1, si0, si1):
    c = lax.axis_index("c")
    s = lax.axis_index("s")
    wid = c * _NS + s
    stripe0 = s * _STRIPE
    idxs = (idx0, idx1)
    ars = (ar0, ar1)
    brs = (br0, br1)
    sas = (sa0, sa1)
    sbs = (sb0, sb1)
    sss = (ss0, ss1)
    sis = (si0, si1)

    zero16 = jnp.zeros((_LANES,), jnp.float32)

    def _zb(i, _):
        for j in range(_H // _LANES):
            ar0[i, pl.ds(j * _LANES, _LANES)] = zero16
            br1[i, pl.ds(j * _LANES, _LANES)] = zero16
        return 0
    lax.fori_loop(0, _K, _zb, 0)

    for blk in range(_STRIPE // _K):
        pltpu.sync_copy(ar0, s_sh.at[pl.ds(stripe0 + blk * _K, _K)])
    plsc.subcore_barrier()

    g0 = wid * _NCHUNK

    def _scatter_wait(buf):
        pltpu.make_async_copy(
            ars[buf], s_sh.at[idxs[buf].at[0]], sss[buf]).wait()

    def _consume(buf):
        pltpu.make_async_copy(a_hbm.at[idxs[buf].at[0]], ars[buf], sas[buf]).wait()
        pltpu.make_async_copy(b_hbm.at[idxs[buf].at[1]], brs[buf], sbs[buf]).wait()
        ar, br = ars[buf], brs[buf]

        def _row(k2, _):
            for r2 in range(2):
                for j in range(_H // _LANES):
                    sl = pl.ds(j * _LANES, _LANES)
                    k = k2 * 2 + r2
                    ar[k, sl] = jnp.maximum(ar[k, sl] + br[k, sl], 0.0)
            return 0
        lax.fori_loop(0, _K // 2, _row, 0)
        pltpu.async_copy(ar, s_sh.at[idxs[buf].at[0]], sss[buf], add=True)

    pltpu.sync_copy(p_hbm.at[g0], idx0)
    pltpu.sync_copy(p_hbm.at[g0], idx1)
    pltpu.async_copy(ar0, s_sh.at[idx0.at[0]], ss0, add=True)
    pltpu.async_copy(br1, s_sh.at[idx1.at[0]], ss1, add=True)
    _scatter_wait(0)
    pltpu.async_copy(a_hbm.at[idx0.at[0]], ar0, sa0)
    pltpu.async_copy(b_hbm.at[idx0.at[1]], br0, sb0)

    def _iter(j, b):
        o = 1 - b
        _scatter_wait(o)
        pltpu.async_copy(p_hbm.at[g0 + j + 1], idxs[o], sis[o])
        _consume(b)
        pltpu.make_async_copy(p_hbm.at[g0 + j + 1], idxs[o], sis[o]).wait()
        pltpu.async_copy(a_hbm.at[idxs[o].at[0]], ars[o], sas[o])
        pltpu.async_copy(b_hbm.at[idxs[o].at[1]], brs[o], sbs[o])

    def _pair(t, _):
        for b in range(2):
            _iter(2 * t + b, b)
        return 0
    lax.fori_loop(0, (_NCHUNK - 1) // 2, _pair, 0)
    _consume((_NCHUNK - 1) % 2)
    _scatter_wait(0)
    _scatter_wait(1)
    plsc.subcore_barrier()

    for blk in range(_STRIPE // _K):
        r0 = stripe0 + blk * _K
        pltpu.sync_copy(s_sh.at[pl.ds(r0, _K)], ar0)
        pltpu.sync_copy(ar0, s_out.at[c, pl.ds(r0, _K)])


@jax.jit
def _edge_stage(a, b, packed_idx):
    mesh = plsc.VectorSubcoreMesh(core_axis_name="c", subcore_axis_name="s")
    return pl.kernel(
        _edge_body,
        out_type=jax.ShapeDtypeStruct((_NC, _NPAD, _H), jnp.float32),
        mesh=mesh,
        scratch_types=[
            pltpu.VMEM_SHARED((_NPAD, _H), jnp.float32),
            pltpu.VMEM((2, _K), jnp.int32),
            pltpu.VMEM((2, _K), jnp.int32),
            pltpu.VMEM((_K, _H), jnp.float32),
            pltpu.VMEM((_K, _H), jnp.float32),
            pltpu.VMEM((_K, _H), jnp.float32),
            pltpu.VMEM((_K, _H), jnp.float32),
            pltpu.SemaphoreType.DMA,
            pltpu.SemaphoreType.DMA,
            pltpu.SemaphoreType.DMA,
            pltpu.SemaphoreType.DMA,
            pltpu.SemaphoreType.DMA,
            pltpu.SemaphoreType.DMA,
            pltpu.SemaphoreType.DMA,
            pltpu.SemaphoreType.DMA,
        ],
    )(a, b, packed_idx)


_ROWS = 2000
_GRID = _N // _ROWS
_PREC = jax.lax.Precision.HIGHEST


def _dot(x, w):
    return jnp.dot(x, w, preferred_element_type=jnp.float32, precision=_PREC)


def _pre_body(x, eW1, eb1, eW2, eb2, mW1t, mb1, mW1b, h0, a, b):
    h = _dot(jnp.maximum(_dot(x[...], eW1[...]) + eb1[...], 0.0), eW2[...]) + eb2[...]
    h0[...] = h
    a[...] = _dot(h, mW1t[...]) + mb1[...]
    b[...] = _dot(h, mW1b[...])


def _mid_body(h, S, mW2, uW1a, uW1b, ub1, uW2, ub2, mW1t, mb1, mW1b,
              h1, a, b):
    hx = h[...]
    agg = _dot(S[0] + S[1], mW2[...])
    u = jnp.maximum(_dot(hx, uW1a[...]) + _dot(agg, uW1b[...]) + ub1[...], 0.0)
    hn = _dot(u, uW2[...]) + ub2[...]
    h1[...] = hn
    a[...] = _dot(hn, mW1t[...]) + mb1[...]
    b[...] = _dot(hn, mW1b[...])


def _post_body(h, S, mW2, uW1a, uW1b, ub1, uW2, ub2, hW1, hb1, hW2, hb2,
               out):
    hx = h[...]
    agg = _dot(S[0] + S[1], mW2[...])
    u = jnp.maximum(_dot(hx, uW1a[...]) + _dot(agg, uW1b[...]) + ub1[...], 0.0)
    hn = _dot(u, uW2[...]) + ub2[...]
    out[...] = _dot(jnp.maximum(_dot(hn, hW1[...]) + hb1[...], 0.0), hW2[...]) + hb2[...]


def _rows_spec(width=_H):
    return pl.BlockSpec((_ROWS, width), lambda i: (i, 0))


def _full_spec(shape):
    nd = len(shape)
    return pl.BlockSpec(shape, lambda i, nd=nd: (0,) * nd)


def _s_spec():
    return pl.BlockSpec((_NC, _ROWS, _H), lambda i: (0, i, 0))


def _w(shape=( _H, _H)):
    return _full_spec(shape)


def _pre_stage(x, eW1, eb1, eW2, eb2, mW1t, mb1, mW1b):
    outs = [jax.ShapeDtypeStruct((_N, _H), jnp.float32)] * 3
    return pl.pallas_call(
        _pre_body,
        grid=(_GRID,),
        in_specs=[_rows_spec(), _w(), _w((1, _H)), _w(), _w((1, _H)),
                  _w(), _w((1, _H)), _w()],
        out_specs=[_rows_spec()] * 3,
        out_shape=outs,
    )(x, eW1, eb1, eW2, eb2, mW1t, mb1, mW1b)


def _mid_stage(h, S, mW2, uW1a, uW1b, ub1, uW2, ub2, mW1t, mb1, mW1b):
    outs = [jax.ShapeDtypeStruct((_N, _H), jnp.float32)] * 3
    return pl.pallas_call(
        _mid_body,
        grid=(_GRID,),
        in_specs=[_rows_spec(), _s_spec(),
                  _w(), _w(), _w(), _w((1, _H)), _w(), _w((1, _H)),
                  _w(), _w((1, _H)), _w()],
        out_specs=[_rows_spec()] * 3,
        out_shape=outs,
    )(h, S, mW2, uW1a, uW1b, ub1, uW2, ub2, mW1t, mb1, mW1b)


def _post_stage(h, S, mW2, uW1a, uW1b, ub1, uW2, ub2, hW1, hb1, hW2, hb2):
    return pl.pallas_call(
        _post_body,
        grid=(_GRID,),
        in_specs=[_rows_spec(), _s_spec(),
                  _w(), _w(), _w(), _w((1, _H)), _w(), _w((1, _H)),
                  _w(), _w((1, _H)), _w(), _w((1, _H))],
        out_specs=_rows_spec(),
        out_shape=jax.ShapeDtypeStruct((_N, _H), jnp.float32),
    )(h, S, mW2, uW1a, uW1b, ub1, uW2, ub2, hW1, hb1, hW2, hb2)


def kernel(x, edge_index, eW1, eb1, eW2, eb2, mW1, mb1, mW2, mb2,
           uW1, ub1, uW2, ub2, hW1, hb1, hW2, hb2):
    src = edge_index[0]
    dst = edge_index[1]
    packed = jnp.stack([dst.reshape(_E // _K, _K), src.reshape(_E // _K, _K)],
                       axis=1)
    r = lambda v: v.reshape(1, -1)

    h0, a1, b1 = _pre_stage(x, eW1, r(eb1), eW2, r(eb2),
                            mW1[0, :_H], r(mb1[0]), mW1[0, _H:])
    s1 = _edge_stage(a1, b1, packed)[:, :_N]
    h1, a2, b2 = _mid_stage(h0, s1, mW2[0],
                            uW1[0, :_H], uW1[0, _H:], r(ub1[0]), uW2[0], r(ub2[0]),
                            mW1[1, :_H], r(mb1[1]), mW1[1, _H:])
    s2 = _edge_stage(a2, b2, packed)[:, :_N]
    return _post_stage(h1, s2, mW2[1],
                       uW1[1, :_H], uW1[1, _H:], r(ub1[1]), uW2[1], r(ub2[1]),
                       hW1, r(hb1), hW2, r(hb2))


# speedup vs baseline: 1.1779x; 1.1779x over previous
import functools

import jax
import jax.numpy as jnp
from jax import lax
from jax.experimental import pallas as pl
from jax.experimental.pallas import tpu as pltpu
from jax.experimental.pallas import tpu_sc as plsc

_N = 10000
_E = 320000
_H = 128
_LANES = 16
_NC = 2
_NS = 16
_NW = _NC * _NS
_EPT = _E // _NW
_K = 80
_NCHUNK = _EPT // _K
_NPAD = 10240
_STRIPE = _NPAD // _NS
_ZROWS = 128
def _edge_body(a_hbm, b_hbm, p_hbm,
               s_out,
               s_sh,
               idx0, idx1,
               ar0, ar1, br0, br1,
               sa0, sa1, sb0, sb1, ss0, ss1):
    c = lax.axis_index("c")
    s = lax.axis_index("s")
    wid = c * _NS + s
    stripe0 = s * _STRIPE
    idxs = (idx0, idx1)
    ars = (ar0, ar1)
    brs = (br0, br1)
    sas = (sa0, sa1)
    sbs = (sb0, sb1)
    sss = (ss0, ss1)

    zero16 = jnp.zeros((_LANES,), jnp.float32)

    def _zb(i, _):
        for j in range(_H // _LANES):
            ar0[i, pl.ds(j * _LANES, _LANES)] = zero16
            br1[i, pl.ds(j * _LANES, _LANES)] = zero16
        return 0
    lax.fori_loop(0, _K, _zb, 0)

    for blk in range(_STRIPE // _K):
        pltpu.sync_copy(ar0, s_sh.at[pl.ds(stripe0 + blk * _K, _K)])
    plsc.subcore_barrier()

    g0 = wid * _NCHUNK

    def _scatter_wait(buf):
        pltpu.make_async_copy(
            ars[buf], s_sh.at[idxs[buf].at[0]], sss[buf]).wait()

    def _issue(g, buf):
        _scatter_wait(buf)
        pltpu.sync_copy(p_hbm.at[g0 + g], idxs[buf])
        pltpu.async_copy(a_hbm.at[idxs[buf].at[0]], ars[buf], sas[buf])
        pltpu.async_copy(b_hbm.at[idxs[buf].at[1]], brs[buf], sbs[buf])

    def _consume(buf):
        pltpu.make_async_copy(a_hbm.at[idxs[buf].at[0]], ars[buf], sas[buf]).wait()
        pltpu.make_async_copy(b_hbm.at[idxs[buf].at[1]], brs[buf], sbs[buf]).wait()
        ar, br = ars[buf], brs[buf]

        def _row(k2, _):
            for r2 in range(2):
                for j in range(_H // _LANES):
                    sl = pl.ds(j * _LANES, _LANES)
                    k = k2 * 2 + r2
                    ar[k, sl] = jnp.maximum(ar[k, sl] + br[k, sl], 0.0)
            return 0
        lax.fori_loop(0, _K // 2, _row, 0)
        pltpu.async_copy(ar, s_sh.at[idxs[buf].at[0]], sss[buf], add=True)

    pltpu.sync_copy(p_hbm.at[g0], idx0)
    pltpu.sync_copy(p_hbm.at[g0], idx1)
    pltpu.async_copy(ar0, s_sh.at[idx0.at[0]], ss0, add=True)
    pltpu.async_copy(br1, s_sh.at[idx1.at[0]], ss1, add=True)
    _issue(0, 0)

    def _pair(t, _):
        for b in range(2):
            j = 2 * t + b
            _issue(j + 1, 1 - b)
            _consume(b)
        return 0
    lax.fori_loop(0, (_NCHUNK - 1) // 2, _pair, 0)
    _consume((_NCHUNK - 1) % 2)
    _scatter_wait(0)
    _scatter_wait(1)
    plsc.subcore_barrier()

    for blk in range(_STRIPE // _K):
        r0 = stripe0 + blk * _K
        pltpu.sync_copy(s_sh.at[pl.ds(r0, _K)], ar0)
        pltpu.sync_copy(ar0, s_out.at[c, pl.ds(r0, _K)])


@jax.jit
def _edge_stage(a, b, packed_idx):
    mesh = plsc.VectorSubcoreMesh(core_axis_name="c", subcore_axis_name="s")
    return pl.kernel(
        _edge_body,
        out_type=jax.ShapeDtypeStruct((_NC, _NPAD, _H), jnp.float32),
        mesh=mesh,
        scratch_types=[
            pltpu.VMEM_SHARED((_NPAD, _H), jnp.float32),
            pltpu.VMEM((2, _K), jnp.int32),
            pltpu.VMEM((2, _K), jnp.int32),
            pltpu.VMEM((_K, _H), jnp.float32),
            pltpu.VMEM((_K, _H), jnp.float32),
            pltpu.VMEM((_K, _H), jnp.float32),
            pltpu.VMEM((_K, _H), jnp.float32),
            pltpu.SemaphoreType.DMA,
            pltpu.SemaphoreType.DMA,
            pltpu.SemaphoreType.DMA,
            pltpu.SemaphoreType.DMA,
            pltpu.SemaphoreType.DMA,
            pltpu.SemaphoreType.DMA,
        ],
    )(a, b, packed_idx)


_ROWS = 2000
_GRID = _N // _ROWS
_PREC = jax.lax.Precision.HIGHEST


def _dot(x, w):
    return jnp.dot(x, w, preferred_element_type=jnp.float32, precision=_PREC)


def _pre_body(x, eW1, eb1, eW2, eb2, mW1t, mb1, mW1b, h0, a, b):
    h = _dot(jnp.maximum(_dot(x[...], eW1[...]) + eb1[...], 0.0), eW2[...]) + eb2[...]
    h0[...] = h
    a[...] = _dot(h, mW1t[...]) + mb1[...]
    b[...] = _dot(h, mW1b[...])


def _mid_body(h, S, mW2, uW1a, uW1b, ub1, uW2, ub2, mW1t, mb1, mW1b,
              h1, a, b):
    hx = h[...]
    agg = _dot(S[0] + S[1], mW2[...])
    u = jnp.maximum(_dot(hx, uW1a[...]) + _dot(agg, uW1b[...]) + ub1[...], 0.0)
    hn = _dot(u, uW2[...]) + ub2[...]
    h1[...] = hn
    a[...] = _dot(hn, mW1t[...]) + mb1[...]
    b[...] = _dot(hn, mW1b[...])


def _post_body(h, S, mW2, uW1a, uW1b, ub1, uW2, ub2, hW1, hb1, hW2, hb2,
               out):
    hx = h[...]
    agg = _dot(S[0] + S[1], mW2[...])
    u = jnp.maximum(_dot(hx, uW1a[...]) + _dot(agg, uW1b[...]) + ub1[...], 0.0)
    hn = _dot(u, uW2[...]) + ub2[...]
    out[...] = _dot(jnp.maximum(_dot(hn, hW1[...]) + hb1[...], 0.0), hW2[...]) + hb2[...]


def _rows_spec(width=_H):
    return pl.BlockSpec((_ROWS, width), lambda i: (i, 0))


def _full_spec(shape):
    nd = len(shape)
    return pl.BlockSpec(shape, lambda i, nd=nd: (0,) * nd)


def _s_spec():
    return pl.BlockSpec((_NC, _ROWS, _H), lambda i: (0, i, 0))


def _w(shape=( _H, _H)):
    return _full_spec(shape)


def _pre_stage(x, eW1, eb1, eW2, eb2, mW1t, mb1, mW1b):
    outs = [jax.ShapeDtypeStruct((_N, _H), jnp.float32)] * 3
    return pl.pallas_call(
        _pre_body,
        grid=(_GRID,),
        in_specs=[_rows_spec(), _w(), _w((1, _H)), _w(), _w((1, _H)),
                  _w(), _w((1, _H)), _w()],
        out_specs=[_rows_spec()] * 3,
        out_shape=outs,
    )(x, eW1, eb1, eW2, eb2, mW1t, mb1, mW1b)


def _mid_stage(h, S, mW2, uW1a, uW1b, ub1, uW2, ub2, mW1t, mb1, mW1b):
    outs = [jax.ShapeDtypeStruct((_N, _H), jnp.float32)] * 3
    return pl.pallas_call(
        _mid_body,
        grid=(_GRID,),
        in_specs=[_rows_spec(), _s_spec(),
                  _w(), _w(), _w(), _w((1, _H)), _w(), _w((1, _H)),
                  _w(), _w((1, _H)), _w()],
        out_specs=[_rows_spec()] * 3,
        out_shape=outs,
    )(h, S, mW2, uW1a, uW1b, ub1, uW2, ub2, mW1t, mb1, mW1b)


def _post_stage(h, S, mW2, uW1a, uW1b, ub1, uW2, ub2, hW1, hb1, hW2, hb2):
    return pl.pallas_call(
        _post_body,
        grid=(_GRID,),
        in_specs=[_rows_spec(), _s_spec(),
                  _w(), _w(), _w(), _w((1, _H)), _w(), _w((1, _H)),
                  _w(), _w((1, _H)), _w(), _w((1, _H))],
        out_specs=_rows_spec(),
        out_shape=jax.ShapeDtypeStruct((_N, _H), jnp.float32),
    )(h, S, mW2, uW1a, uW1b, ub1, uW2, ub2, hW1, hb1, hW2, hb2)


def kernel(x, edge_index, eW1, eb1, eW2, eb2, mW1, mb1, mW2, mb2,
           uW1, ub1, uW2, ub2, hW1, hb1, hW2, hb2):
    src = edge_index[0]
    dst = edge_index[1]
    packed = jnp.stack([dst.reshape(_E // _K, _K), src.reshape(_E // _K, _K)],
                       axis=1)
    r = lambda v: v.reshape(1, -1)

    h0, a1, b1 = _pre_stage(x, eW1, r(eb1), eW2, r(eb2),
                            mW1[0, :_H], r(mb1[0]), mW1[0, _H:])
    s1 = _edge_stage(a1, b1, packed)[:, :_N]
    h1, a2, b2 = _mid_stage(h0, s1, mW2[0],
                            uW1[0, :_H], uW1[0, _H:], r(ub1[0]), uW2[0], r(ub2[0]),
                            mW1[1, :_H], r(mb1[1]), mW1[1, _H:])
    s2 = _edge_stage(a2, b2, packed)[:, :_N]
    return _post_stage(h1, s2, mW2[1],
                       uW1[1, :_H], uW1[1, _H:], r(ub1[1]), uW2[1], r(ub2[1]),
                       hW1, r(hb1), hW2, r(hb2))


# speedup vs baseline: 1.2027x; 1.0210x over previous
import functools

import jax
import jax.numpy as jnp
from jax import lax
from jax.experimental import pallas as pl
from jax.experimental.pallas import tpu as pltpu
from jax.experimental.pallas import tpu_sc as plsc

_N = 10000
_E = 320000
_H = 128
_LANES = 16
_NC = 2
_NS = 16
_NW = _NC * _NS
_EPT = _E // _NW
_K = 80
_NCHUNK = _EPT // _K
_NPAD = 10240
_STRIPE = _NPAD // _NS
_ZROWS = 128
def _edge_body(a_hbm, b_hbm, p_hbm,
               s_out,
               s_sh,
               idx0, idx1,
               ar0, ar1, br0, br1,
               sa0, sa1, sb0, sb1, ss0, ss1):
    c = lax.axis_index("c")
    s = lax.axis_index("s")
    wid = c * _NS + s
    stripe0 = s * _STRIPE
    idxs = (idx0, idx1)
    ars = (ar0, ar1)
    brs = (br0, br1)
    sas = (sa0, sa1)
    sbs = (sb0, sb1)
    sss = (ss0, ss1)

    zero16 = jnp.zeros((_LANES,), jnp.float32)

    def _zb(i, _):
        for j in range(_H // _LANES):
            ar0[i, pl.ds(j * _LANES, _LANES)] = zero16
            br1[i, pl.ds(j * _LANES, _LANES)] = zero16
        return 0
    lax.fori_loop(0, _K, _zb, 0)

    for blk in range(_STRIPE // _K):
        pltpu.sync_copy(ar0, s_sh.at[pl.ds(stripe0 + blk * _K, _K)])
    plsc.subcore_barrier()

    g0 = wid * _NCHUNK

    def _scatter_wait(buf):
        pltpu.make_async_copy(
            ars[buf], s_sh.at[idxs[buf].at[0]], sss[buf]).wait()

    def _issue(g, buf):
        _scatter_wait(buf)
        pltpu.sync_copy(p_hbm.at[g0 + g], idxs[buf])
        pltpu.async_copy(a_hbm.at[idxs[buf].at[0]], ars[buf], sas[buf])
        pltpu.async_copy(b_hbm.at[idxs[buf].at[1]], brs[buf], sbs[buf])

    def _consume(buf):
        pltpu.make_async_copy(a_hbm.at[idxs[buf].at[0]], ars[buf], sas[buf]).wait()
        pltpu.make_async_copy(b_hbm.at[idxs[buf].at[1]], brs[buf], sbs[buf]).wait()
        ar, br = ars[buf], brs[buf]

        def _row(k2, _):
            for r2 in range(2):
                for j in range(_H // _LANES):
                    sl = pl.ds(j * _LANES, _LANES)
                    k = k2 * 2 + r2
                    ar[k, sl] = jnp.maximum(ar[k, sl] + br[k, sl], 0.0)
            return 0
        lax.fori_loop(0, _K // 2, _row, 0)
        pltpu.async_copy(ar, s_sh.at[idxs[buf].at[0]], sss[buf], add=True)

    pltpu.sync_copy(p_hbm.at[g0], idx0)
    pltpu.sync_copy(p_hbm.at[g0], idx1)
    pltpu.async_copy(ar0, s_sh.at[idx0.at[0]], ss0, add=True)
    pltpu.async_copy(br1, s_sh.at[idx1.at[0]], ss1, add=True)
    _issue(0, 0)

    def _pair(t, _):
        for b in range(2):
            j = 2 * t + b
            _issue(j + 1, 1 - b)
            _consume(b)
        return 0
    lax.fori_loop(0, (_NCHUNK - 1) // 2, _pair, 0)
    _consume((_NCHUNK - 1) % 2)
    _scatter_wait(0)
    _scatter_wait(1)
    plsc.subcore_barrier()

    for blk in range(_STRIPE // _K):
        r0 = stripe0 + blk * _K
        pltpu.sync_copy(s_sh.at[pl.ds(r0, _K)], ar0)
        pltpu.sync_copy(ar0, s_out.at[c, pl.ds(r0, _K)])


@jax.jit
def _edge_stage(a, b, packed_idx):
    mesh = plsc.VectorSubcoreMesh(core_axis_name="c", subcore_axis_name="s")
    return pl.kernel(
        _edge_body,
        out_type=jax.ShapeDtypeStruct((_NC, _NPAD, _H), jnp.float32),
        mesh=mesh,
        scratch_types=[
            pltpu.VMEM_SHARED((_NPAD, _H), jnp.float32),
            pltpu.VMEM((2, _K), jnp.int32),
            pltpu.VMEM((2, _K), jnp.int32),
            pltpu.VMEM((_K, _H), jnp.float32),
            pltpu.VMEM((_K, _H), jnp.float32),
            pltpu.VMEM((_K, _H), jnp.float32),
            pltpu.VMEM((_K, _H), jnp.float32),
            pltpu.SemaphoreType.DMA,
            pltpu.SemaphoreType.DMA,
            pltpu.SemaphoreType.DMA,
            pltpu.SemaphoreType.DMA,
            pltpu.SemaphoreType.DMA,
            pltpu.SemaphoreType.DMA,
        ],
    )(a, b, packed_idx)


_ROWS = 2000
_GRID = _N // _ROWS
_PREC = jax.lax.Precision.HIGHEST


def _dot(x, w):
    return jnp.dot(x, w, preferred_element_type=jnp.float32, precision=_PREC)


def _pre_body(x, eW1, eb1, eW2, eb2, mW1t, mb1, mW1b, h0, a, b):
    h = _dot(jnp.maximum(_dot(x[...], eW1[...]) + eb1[...], 0.0), eW2[...]) + eb2[...]
    h0[...] = h
    a[...] = _dot(h, mW1t[...]) + mb1[...]
    b[...] = _dot(h, mW1b[...])


def _mid_body(h, S, mW2, uW1a, uW1b, ub1, uW2, ub2, mW1t, mb1, mW1b,
              h1, a, b):
    hx = h[...]
    agg = _dot(S[0] + S[1], mW2[...])
    u = jnp.maximum(_dot(hx, uW1a[...]) + _dot(agg, uW1b[...]) + ub1[...], 0.0)
    hn = _dot(u, uW2[...]) + ub2[...]
    h1[...] = hn
    a[...] = _dot(hn, mW1t[...]) + mb1[...]
    b[...] = _dot(hn, mW1b[...])


def _post_body(h, S, mW2, uW1a, uW1b, ub1, uW2, ub2, hW1, hb1, hW2, hb2,
               out):
    hx = h[...]
    agg = _dot(S[0] + S[1], mW2[...])
    u = jnp.maximum(_dot(hx, uW1a[...]) + _dot(agg, uW1b[...]) + ub1[...], 0.0)
    hn = _dot(u, uW2[...]) + ub2[...]
    out[...] = _dot(jnp.maximum(_dot(hn, hW1[...]) + hb1[...], 0.0), hW2[...]) + hb2[...]


def _rows_spec(width=_H):
    return pl.BlockSpec((_ROWS, width), lambda i: (i, 0))


def _full_spec(shape):
    nd = len(shape)
    return pl.BlockSpec(shape, lambda i, nd=nd: (0,) * nd)


def _s_spec():
    return pl.BlockSpec((_NC, _ROWS, _H), lambda i: (0, i, 0))


def _w(shape=( _H, _H)):
    return _full_spec(shape)


def _pre_stage(x, eW1, eb1, eW2, eb2, mW1t, mb1, mW1b):
    outs = [jax.ShapeDtypeStruct((_N, _H), jnp.float32)] * 3
    return pl.pallas_call(
        _pre_body,
        grid=(_GRID,),
        in_specs=[_rows_spec(), _w(), _w((1, _H)), _w(), _w((1, _H)),
                  _w(), _w((1, _H)), _w()],
        out_specs=[_rows_spec()] * 3,
        out_shape=outs,
    )(x, eW1, eb1, eW2, eb2, mW1t, mb1, mW1b)


def _mid_stage(h, S, mW2, uW1a, uW1b, ub1, uW2, ub2, mW1t, mb1, mW1b):
    outs = [jax.ShapeDtypeStruct((_N, _H), jnp.float32)] * 3
    return pl.pallas_call(
        _mid_body,
        grid=(_GRID,),
        in_specs=[_rows_spec(), _s_spec(),
                  _w(), _w(), _w(), _w((1, _H)), _w(), _w((1, _H)),
                  _w(), _w((1, _H)), _w()],
        out_specs=[_rows_spec()] * 3,
        out_shape=outs,
    )(h, S, mW2, uW1a, uW1b, ub1, uW2, ub2, mW1t, mb1, mW1b)


def _post_stage(h, S, mW2, uW1a, uW1b, ub1, uW2, ub2, hW1, hb1, hW2, hb2):
    return pl.pallas_call(
        _post_body,
        grid=(_GRID,),
        in_specs=[_rows_spec(), _s_spec(),
                  _w(), _w(), _w(), _w((1, _H)), _w(), _w((1, _H)),
                  _w(), _w((1, _H)), _w(), _w((1, _H))],
        out_specs=_rows_spec(),
        out_shape=jax.ShapeDtypeStruct((_N, _H), jnp.float32),
    )(h, S, mW2, uW1a, uW1b, ub1, uW2, ub2, hW1, hb1, hW2, hb2)


def kernel(x, edge_index, eW1, eb1, eW2, eb2, mW1, mb1, mW2, mb2,
           uW1, ub1, uW2, ub2, hW1, hb1, hW2, hb2):
    src = edge_index[0]
    dst = edge_index[1]
    packed = jnp.stack([dst.reshape(_E // _K, _K), src.reshape(_E // _K, _K)],
                       axis=1)
    r = lambda v: v.reshape(1, -1)

    h0, a1, b1 = _pre_stage(x, eW1, r(eb1), eW2, r(eb2),
                            mW1[0, :_H], r(mb1[0]), mW1[0, _H:])
    s1 = _edge_stage(a1, b1, packed)
    h1, a2, b2 = _mid_stage(h0, s1, mW2[0],
                            uW1[0, :_H], uW1[0, _H:], r(ub1[0]), uW2[0], r(ub2[0]),
                            mW1[1, :_H], r(mb1[1]), mW1[1, _H:])
    s2 = _edge_stage(a2, b2, packed)
    return _post_stage(h1, s2, mW2[1],
                       uW1[1, :_H], uW1[1, _H:], r(ub1[1]), uW2[1], r(ub2[1]),
                       hW1, r(hb1), hW2, r(hb2))
